# baseline (speedup 1.0000x reference)
import functools

import jax
import jax.numpy as jnp
from jax import lax
from jax.experimental import pallas as pl
from jax.experimental.pallas import tpu as pltpu
from jax.experimental.pallas import tpu_sc as plsc

N_NODES = 10000
N_EDGES = 160000
BATCH = 256

L = 16
NC = 2
NS = 16
NW = NC * NS
ROWS = BATCH // NW
PAIRS = ROWS // 2
CHUNK = 3200
N_CHUNKS = N_EDGES // CHUNK


def _energy_body(packed_hbm, ii_hbm, jj_hbm, lin_hbm, q_hbm, out_hbm,
                 s_v, lin_v, iv_v, jv_v, qv_v, ob_v, sem0, sem1):
    wid = lax.axis_index("s") * NC + lax.axis_index("c")
    base = wid * (PAIRS * N_NODES)

    sems = (sem0, sem1)

    def fire(off, slot):
        pltpu.async_copy(ii_hbm.at[pl.ds(off, CHUNK)], iv_v.at[slot], sems[slot])
        pltpu.async_copy(jj_hbm.at[pl.ds(off, CHUNK)], jv_v.at[slot], sems[slot])
        pltpu.async_copy(q_hbm.at[pl.ds(off, CHUNK)], qv_v.at[slot], sems[slot])

    def drain(slot):
        pltpu.make_async_copy(ii_hbm.at[pl.ds(0, CHUNK)], iv_v.at[slot], sems[slot]).wait()
        pltpu.make_async_copy(jj_hbm.at[pl.ds(0, CHUNK)], jv_v.at[slot], sems[slot]).wait()
        pltpu.make_async_copy(q_hbm.at[pl.ds(0, CHUNK)], qv_v.at[slot], sems[slot]).wait()

    fire(0 * CHUNK, 0)
    fire(1 * CHUNK, 1)

    pltpu.sync_copy(packed_hbm.at[pl.ds(base, PAIRS * N_NODES)], s_v)
    pltpu.sync_copy(lin_hbm, lin_v)

    accs0 = tuple(jnp.zeros((L,), jnp.float32) for _ in range(ROWS))

    @plsc.parallel_loop(0, N_NODES // L, unroll=2, carry=accs0)
    def accs(v, accs):
        lv = lin_v[pl.ds(v * L, L)]
        new = list(accs)
        for p in range(PAIRS):
            w = plsc.bitcast(s_v[pl.ds(p * N_NODES + v * L, L)], jnp.bfloat16)
            slo, shi = plsc.unpack(w, format=plsc.PackFormat.INTERLEAVED)
            new[2 * p] = new[2 * p] + slo * lv
            new[2 * p + 1] = new[2 * p + 1] + shi * lv
        return tuple(new)

    def chunk_pair(g, accs):
        for slot in range(2):
            c = g * 2 + slot
            drain(slot)

            @plsc.parallel_loop(0, CHUNK // L, unroll=4, carry=accs)
            def accs(v, accs):
                iv = iv_v[slot, pl.ds(v * L, L)]
                jv = jv_v[slot, pl.ds(v * L, L)]
                qv = qv_v[slot, pl.ds(v * L, L)]
                new = list(accs)
                for p in range(PAIRS):
                    ga = plsc.load_gather(s_v, [iv + p * N_NODES])
                    gb = plsc.load_gather(s_v, [jv + p * N_NODES])
                    prod = plsc.bitcast(ga, jnp.bfloat16) * plsc.bitcast(gb, jnp.bfloat16)
                    lo, hi = plsc.unpack(prod, format=plsc.PackFormat.INTERLEAVED)
                    new[2 * p] = new[2 * p] + qv * lo
                    new[2 * p + 1] = new[2 * p + 1] + qv * hi
                return tuple(new)

            @pl.when(c + 2 < N_CHUNKS)
            def _():
                fire((c + 2) * CHUNK, slot)
        return accs

    accs = lax.fori_loop(0, N_CHUNKS // 2, chunk_pair, accs)

    lane = lax.iota(jnp.int32, L)
    ob = jnp.zeros((L,), jnp.float32)
    for r in range(ROWS):
        ob = jnp.where(lane == r, jnp.sum(accs[r]), ob)
    ob_v[...] = ob
    pltpu.sync_copy(ob_v, out_hbm.at[wid])


_energy_kernel = functools.partial(
    pl.kernel,
    out_type=jax.ShapeDtypeStruct((NW, L), jnp.float32),
    mesh=plsc.VectorSubcoreMesh(core_axis_name="c", subcore_axis_name="s"),
    compiler_params=pltpu.CompilerParams(needs_layout_passes=False),
    scratch_types=[
        pltpu.VMEM((PAIRS * N_NODES,), jnp.int32),
        pltpu.VMEM((N_NODES,), jnp.float32),
        pltpu.VMEM((2, CHUNK), jnp.int32),
        pltpu.VMEM((2, CHUNK), jnp.int32),
        pltpu.VMEM((2, CHUNK), jnp.float32),
        pltpu.VMEM((L,), jnp.float32),
        pltpu.SemaphoreType.DMA,
        pltpu.SemaphoreType.DMA,
    ],
)(_energy_body)


def kernel(spins, edge_idx_i, edge_idx_j, linear, quadratic):
    u = jax.lax.bitcast_convert_type(spins.astype(jnp.bfloat16), jnp.uint16)
    u = u.astype(jnp.uint32).reshape(BATCH // 2, 2, N_NODES)
    packed = jax.lax.bitcast_convert_type(u[:, 0] | (u[:, 1] << 16), jnp.int32)
    out2d = _energy_kernel(packed.reshape(-1), edge_idx_i.astype(jnp.int32),
                           edge_idx_j.astype(jnp.int32), linear, quadratic)
    return out2d[:, :ROWS].reshape(BATCH)


# speedup vs baseline: 1.4480x; 1.4480x over previous
import functools

import jax
import jax.numpy as jnp
from jax import lax
from jax.experimental import pallas as pl
from jax.experimental.pallas import tpu as pltpu
from jax.experimental.pallas import tpu_sc as plsc

N_NODES = 10000
N_EDGES = 160000
BATCH = 256

L = 16
NC = 2
NS = 16
NW = NC * NS
ROWS = BATCH // NW
PAIRS = ROWS // 2
CHUNK = 3200
N_CHUNKS = N_EDGES // CHUNK


def _energy_body(spins_hbm, ii_hbm, jj_hbm, lin_hbm, q_hbm, out_hbm,
                 pk_v, st_v, lin_v, iv_v, jv_v, qv_v, ob_v, sem0, sem1):
    wid = lax.axis_index("s") * NC + lax.axis_index("c")
    base = wid * (ROWS * N_NODES)

    sems = (sem0, sem1)

    def fire(off, slot):
        pltpu.async_copy(ii_hbm.at[pl.ds(off, CHUNK)], iv_v.at[slot], sems[slot])
        pltpu.async_copy(jj_hbm.at[pl.ds(off, CHUNK)], jv_v.at[slot], sems[slot])
        pltpu.async_copy(q_hbm.at[pl.ds(off, CHUNK)], qv_v.at[slot], sems[slot])

    def drain(slot):
        pltpu.make_async_copy(ii_hbm.at[pl.ds(0, CHUNK)], iv_v.at[slot], sems[slot]).wait()
        pltpu.make_async_copy(jj_hbm.at[pl.ds(0, CHUNK)], jv_v.at[slot], sems[slot]).wait()
        pltpu.make_async_copy(q_hbm.at[pl.ds(0, CHUNK)], qv_v.at[slot], sems[slot]).wait()

    fire(0 * CHUNK, 0)
    fire(1 * CHUNK, 1)

    pltpu.sync_copy(lin_hbm, lin_v)

    zz = (jnp.zeros((L,), jnp.float32), jnp.zeros((L,), jnp.float32))
    accs = []
    for p in range(PAIRS):
        pltpu.sync_copy(
            spins_hbm.at[pl.ds(base + 2 * p * N_NODES, 2 * N_NODES)], st_v)

        @plsc.parallel_loop(0, N_NODES // L, unroll=2, carry=zz)
        def lacc(v, lacc):
            a = st_v[pl.ds(v * L, L)]
            b = st_v[pl.ds(N_NODES + v * L, L)]
            pk_v[pl.ds(p * N_NODES + v * L, L)] = plsc.bitcast(
                plsc.pack(a, b, format=plsc.PackFormat.INTERLEAVED), jnp.int32)
            lv = lin_v[pl.ds(v * L, L)]
            return (lacc[0] + a * lv, lacc[1] + b * lv)

        accs += [lacc[0], lacc[1]]
    accs = tuple(accs)

    def chunk_pair(g, accs):
        for slot in range(2):
            c = g * 2 + slot
            drain(slot)

            @plsc.parallel_loop(0, CHUNK // L, unroll=4, carry=accs)
            def accs(v, accs):
                iv = iv_v[slot, pl.ds(v * L, L)]
                jv = jv_v[slot, pl.ds(v * L, L)]
                qv = qv_v[slot, pl.ds(v * L, L)]
                new = list(accs)
                for p in range(PAIRS):
                    ga = plsc.load_gather(pk_v, [iv + p * N_NODES])
                    gb = plsc.load_gather(pk_v, [jv + p * N_NODES])
                    prod = plsc.bitcast(ga, jnp.bfloat16) * plsc.bitcast(gb, jnp.bfloat16)
                    lo, hi = plsc.unpack(prod, format=plsc.PackFormat.INTERLEAVED)
                    new[2 * p] = new[2 * p] + qv * lo
                    new[2 * p + 1] = new[2 * p + 1] + qv * hi
                return tuple(new)

            @pl.when(c + 2 < N_CHUNKS)
            def _():
                fire((c + 2) * CHUNK, slot)
        return accs

    accs = lax.fori_loop(0, N_CHUNKS // 2, chunk_pair, accs)

    lane = lax.iota(jnp.int32, L)
    ob = jnp.zeros((L,), jnp.float32)
    for r in range(ROWS):
        ob = jnp.where(lane == r, jnp.sum(accs[r]), ob)
    ob_v[...] = ob
    pltpu.sync_copy(ob_v, out_hbm.at[wid])


_energy_kernel = functools.partial(
    pl.kernel,
    out_type=jax.ShapeDtypeStruct((NW, L), jnp.float32),
    mesh=plsc.VectorSubcoreMesh(core_axis_name="c", subcore_axis_name="s"),
    compiler_params=pltpu.CompilerParams(needs_layout_passes=False),
    scratch_types=[
        pltpu.VMEM((PAIRS * N_NODES,), jnp.int32),
        pltpu.VMEM((2 * N_NODES,), jnp.float32),
        pltpu.VMEM((N_NODES,), jnp.float32),
        pltpu.VMEM((2, CHUNK), jnp.int32),
        pltpu.VMEM((2, CHUNK), jnp.int32),
        pltpu.VMEM((2, CHUNK), jnp.float32),
        pltpu.VMEM((L,), jnp.float32),
        pltpu.SemaphoreType.DMA,
        pltpu.SemaphoreType.DMA,
    ],
)(_energy_body)


def kernel(spins, edge_idx_i, edge_idx_j, linear, quadratic):
    out2d = _energy_kernel(spins.reshape(-1), edge_idx_i.astype(jnp.int32),
                           edge_idx_j.astype(jnp.int32), linear, quadratic)
    return out2d[:, :ROWS].reshape(BATCH)
